# baseline (speedup 1.0000x reference)
import jax
import jax.numpy as jnp
from jax.experimental import pallas as pl
from jax.experimental.pallas import tpu as pltpu

E = 16
TILE = 512
D = 2048


def _router_body(x_ref, w_ref, weights_ref, topw_ref, tope_ref):
    x = x_ref[...]
    w = w_ref[...]
    logits = jnp.dot(x, w, preferred_element_type=jnp.float32)
    m = jnp.max(logits, axis=-1, keepdims=True)
    ex = jnp.exp(logits - m)
    s = jnp.sum(ex, axis=-1, keepdims=True)
    weights = ex / s
    weights_ref[...] = weights

    col = jax.lax.broadcasted_iota(jnp.int32, weights.shape, 1)
    w1 = jnp.max(weights, axis=-1, keepdims=True)
    i1 = jnp.min(jnp.where(weights == w1, col, E), axis=-1, keepdims=True)
    masked = jnp.where(col == i1, -jnp.inf, weights)
    w2 = jnp.max(masked, axis=-1, keepdims=True)
    i2 = jnp.min(jnp.where(masked == w2, col, E), axis=-1, keepdims=True)
    norm = w1 + w2
    topw_ref[...] = jnp.concatenate([w1 / norm, w2 / norm], axis=-1)
    tope_ref[...] = jnp.concatenate([i1, i2], axis=-1)


def kernel(x, W):
    B, S, _ = x.shape
    N = B * S
    x2 = x.reshape(N, D)
    grid = (N // TILE,)
    weights, topw, tope = pl.pallas_call(
        _router_body,
        grid=grid,
        in_specs=[
            pl.BlockSpec((TILE, D), lambda i: (i, 0)),
            pl.BlockSpec((D, E), lambda i: (0, 0)),
        ],
        out_specs=[
            pl.BlockSpec((TILE, E), lambda i: (i, 0)),
            pl.BlockSpec((TILE, 2), lambda i: (i, 0)),
            pl.BlockSpec((TILE, 2), lambda i: (i, 0)),
        ],
        out_shape=[
            jax.ShapeDtypeStruct((N, E), jnp.float32),
            jax.ShapeDtypeStruct((N, 2), jnp.float32),
            jax.ShapeDtypeStruct((N, 2), jnp.int32),
        ],
    )(x2, W)
    return (
        weights.reshape(B, S, E),
        topw.reshape(B, S, 2),
        tope.reshape(B, S, 2),
    )


# speedup vs baseline: 1.0943x; 1.0943x over previous
import jax
import jax.numpy as jnp
from jax.experimental import pallas as pl
from jax.experimental.pallas import tpu as pltpu

E = 16
TILE = 512
D = 2048


def _router_body(x_ref, w_ref, weights_ref, topw_ref, tope_ref):
    x = x_ref[...]
    w = w_ref[...]
    lt = jax.lax.dot_general(w, x, (((0,), (1,)), ((), ())),
                             preferred_element_type=jnp.float32)
    m = jnp.max(lt, axis=0, keepdims=True)
    ex = jnp.exp(lt - m)
    s = jnp.sum(ex, axis=0, keepdims=True)
    weights_ref[...] = (ex / s).T

    row = jax.lax.broadcasted_iota(jnp.int32, lt.shape, 0)
    i1 = jnp.min(jnp.where(lt == m, row, E), axis=0, keepdims=True)
    masked = jnp.where(row == i1, -jnp.inf, lt)
    l2 = jnp.max(masked, axis=0, keepdims=True)
    i2 = jnp.min(jnp.where(masked == l2, row, E), axis=0, keepdims=True)
    e2 = jnp.exp(l2 - m)
    r = 1.0 / (1.0 + e2)
    topw_ref[...] = jnp.concatenate([r, e2 * r], axis=0).T
    tope_ref[...] = jnp.concatenate([i1, i2], axis=0).T


def kernel(x, W):
    B, S, _ = x.shape
    N = B * S
    x2 = x.reshape(N, D)
    grid = (N // TILE,)
    weights, topw, tope = pl.pallas_call(
        _router_body,
        grid=grid,
        in_specs=[
            pl.BlockSpec((TILE, D), lambda i: (i, 0)),
            pl.BlockSpec((D, E), lambda i: (0, 0)),
        ],
        out_specs=[
            pl.BlockSpec((TILE, E), lambda i: (i, 0)),
            pl.BlockSpec((TILE, 2), lambda i: (i, 0)),
            pl.BlockSpec((TILE, 2), lambda i: (i, 0)),
        ],
        out_shape=[
            jax.ShapeDtypeStruct((N, E), jnp.float32),
            jax.ShapeDtypeStruct((N, 2), jnp.float32),
            jax.ShapeDtypeStruct((N, 2), jnp.int32),
        ],
    )(x2, W)
    return (
        weights.reshape(B, S, E),
        topw.reshape(B, S, 2),
        tope.reshape(B, S, 2),
    )


# speedup vs baseline: 1.2350x; 1.1286x over previous
import jax
import jax.numpy as jnp
from jax.experimental import pallas as pl
from jax.experimental.pallas import tpu as pltpu

E = 16
TILE = 1024
D = 2048


def _router_body(x_ref, w_ref, weights_ref, topw_ref, tope_ref):
    x = x_ref[...]
    w = w_ref[...]
    lt = jax.lax.dot_general(w, x, (((0,), (1,)), ((), ())),
                             preferred_element_type=jnp.float32)
    m = jnp.max(lt, axis=0, keepdims=True)
    ex = jnp.exp(lt - m)
    s = jnp.sum(ex, axis=0, keepdims=True)
    weights_ref[...] = (ex / s).T

    row = jax.lax.broadcasted_iota(jnp.int32, lt.shape, 0)
    i1 = jnp.min(jnp.where(lt == m, row, E), axis=0, keepdims=True)
    masked = jnp.where(row == i1, -jnp.inf, lt)
    l2 = jnp.max(masked, axis=0, keepdims=True)
    i2 = jnp.min(jnp.where(masked == l2, row, E), axis=0, keepdims=True)
    e2 = jnp.exp(l2 - m)
    r = 1.0 / (1.0 + e2)
    topw_ref[...] = jnp.concatenate([r, e2 * r], axis=0).T
    tope_ref[...] = jnp.concatenate([i1, i2], axis=0).T


def kernel(x, W):
    B, S, _ = x.shape
    N = B * S
    x2 = x.reshape(N, D)
    grid = (N // TILE,)
    weights, topw, tope = pl.pallas_call(
        _router_body,
        grid=grid,
        in_specs=[
            pl.BlockSpec((TILE, D), lambda i: (i, 0)),
            pl.BlockSpec((D, E), lambda i: (0, 0)),
        ],
        out_specs=[
            pl.BlockSpec((TILE, E), lambda i: (i, 0)),
            pl.BlockSpec((TILE, 2), lambda i: (i, 0)),
            pl.BlockSpec((TILE, 2), lambda i: (i, 0)),
        ],
        out_shape=[
            jax.ShapeDtypeStruct((N, E), jnp.float32),
            jax.ShapeDtypeStruct((N, 2), jnp.float32),
            jax.ShapeDtypeStruct((N, 2), jnp.int32),
        ],
    )(x2, W)
    return (
        weights.reshape(B, S, E),
        topw.reshape(B, S, 2),
        tope.reshape(B, S, 2),
    )
